# baseline (speedup 1.0000x reference)
import functools

import numpy as np
import jax
import jax.numpy as jnp
from jax import lax
from jax.experimental import pallas as pl
from jax.experimental.pallas import tpu as pltpu

_S = 128
_P = 136


def _upsample_matrix(n_in, n_out):
    pos = np.arange(n_out) * (n_in - 1) / (n_out - 1)
    lo = np.clip(np.floor(pos).astype(np.int64), 0, n_in - 2)
    frac = (pos - lo).astype(np.float32)
    m = np.zeros((n_out, n_in), np.float32)
    m[np.arange(n_out), lo] += 1.0 - frac
    m[np.arange(n_out), lo + 1] += frac
    return m


def _block_kernel(x_ref, m_ref, w1_ref, b1_ref, w2_ref, bout_ref,
                  o_ref, tb_ref, abuf_ref, p1_ref, hbuf_ref, p2_ref,
                  *, H, W, C, Ho, Wo, Co):
    C2 = 2 * C
    x = x_ref[...]
    xcat = jnp.concatenate([jnp.maximum(x, 0.0), x], axis=2)
    m = m_ref[...]
    t = jnp.dot(m, xcat.reshape(W, H * C2),
                preferred_element_type=jnp.float32)
    tt = jnp.transpose(t.reshape(Wo, H, C2), (1, 0, 2))
    tb_ref[...] = jnp.zeros_like(tb_ref)
    tb_ref[:, :Wo, :] = tt
    u = jnp.dot(m, tb_ref[...].reshape(H, _S * C2),
                preferred_element_type=jnp.float32)
    u = u.reshape(Ho, _S, C2)

    abuf_ref[...] = jnp.zeros_like(abuf_ref)
    abuf_ref[1:1 + Ho, 8:8 + Wo, :] = u[:, :Wo, :C].astype(jnp.bfloat16)
    fa = abuf_ref[...].reshape((Ho + 3) * _S, C)
    np1 = (Ho + 2) * _S
    for kc in range(3):
        p1_ref[:, kc * C:(kc + 1) * C] = fa[7 + kc:7 + kc + np1, :]
    mo = Ho * _S
    acc = jnp.dot(p1_ref[0:mo], w1_ref[0], preferred_element_type=jnp.float32)
    acc = acc + jnp.dot(p1_ref[_S:_S + mo], w1_ref[1],
                        preferred_element_type=jnp.float32)
    acc = acc + jnp.dot(p1_ref[2 * _S:2 * _S + mo], w1_ref[2],
                        preferred_element_type=jnp.float32)
    h = jnp.maximum(acc + b1_ref[...], 0.0).astype(jnp.bfloat16)
    h3 = h.reshape(Ho, _S, Co)
    xmask = lax.broadcasted_iota(jnp.int32, (1, _S, 1), 1) < Wo
    h3 = jnp.where(xmask, h3, jnp.bfloat16(0))
    hbuf_ref[...] = jnp.zeros_like(hbuf_ref)
    hbuf_ref[1:1 + Ho, 8:8 + Wo, :] = h3[:, :Wo, :]
    fh = hbuf_ref[...].reshape((Ho + 3) * _S, Co)

    us = u.reshape(Ho * _S, C2)[:, C:].astype(jnp.bfloat16)
    mh = mo // 2
    for half in range(2):
        r0 = half * mh
        for kr in range(3):
            for kc in range(3):
                blk = kr * 3 + kc
                p2_ref[:, blk * Co:(blk + 1) * Co] = \
                    fh[7 + kr * _S + kc + r0:7 + kr * _S + kc + r0 + mh, :]
        p2_ref[:, 9 * Co:9 * Co + C] = us[r0:r0 + mh, :]
        accT = lax.dot_general(w2_ref[...], p2_ref[...],
                               (((0,), (1,)), ((), ())),
                               preferred_element_type=jnp.float32)
        outT = accT + bout_ref[...]
        o_ref[:, half * (Ho // 2):(half + 1) * (Ho // 2), :] = \
            outT.reshape(Co, Ho // 2, _S)[:, :, :Wo]


def kernel(w1, b1, w2, b2, wsc, bsc, x):
    B, Cin, H, W = x.shape
    Co = w1.shape[-1]
    Ho, Wo = 2 * H, 2 * W
    xt = jnp.transpose(x, (0, 3, 2, 1))
    m = jnp.asarray(_upsample_matrix(H, Ho))
    w1c = w1[:, :, :Cin, :].reshape(3, 3 * Cin, Co).astype(jnp.bfloat16)
    w2c = jnp.concatenate([w2.reshape(9 * Co, Co), wsc[:Cin, :]],
                          axis=0).astype(jnp.bfloat16)
    bout = (b2 + bsc).reshape(Co, 1)

    kern = functools.partial(_block_kernel, H=H, W=W, C=Cin,
                             Ho=Ho, Wo=Wo, Co=Co)
    return pl.pallas_call(
        kern,
        out_shape=jax.ShapeDtypeStruct((B, Co, Ho, Wo), jnp.float32),
        grid_spec=pltpu.PrefetchScalarGridSpec(
            num_scalar_prefetch=0,
            grid=(B,),
            in_specs=[
                pl.BlockSpec((None, W, H, Cin), lambda b: (b, 0, 0, 0)),
                pl.BlockSpec((Ho, H), lambda b: (0, 0)),
                pl.BlockSpec((3, 3 * Cin, Co), lambda b: (0, 0, 0)),
                pl.BlockSpec((1, Co), lambda b: (0, 0)),
                pl.BlockSpec((9 * Co + Cin, Co), lambda b: (0, 0)),
                pl.BlockSpec((Co, 1), lambda b: (0, 0)),
            ],
            out_specs=pl.BlockSpec((None, Co, Ho, Wo), lambda b: (b, 0, 0, 0)),
            scratch_shapes=[
                pltpu.VMEM((H, _S, 2 * Cin), jnp.float32),
                pltpu.VMEM((Ho + 3, _S, Cin), jnp.bfloat16),
                pltpu.VMEM(((Ho + 2) * _S, 3 * Cin), jnp.bfloat16),
                pltpu.VMEM((Ho + 3, _S, Co), jnp.bfloat16),
                pltpu.VMEM((Ho * _S // 2, 9 * Co + Cin), jnp.bfloat16),
            ],
        ),
        compiler_params=pltpu.CompilerParams(dimension_semantics=("parallel",)),
    )(xt, m, w1c, b1.reshape(1, Co), w2c, bout)


# speedup vs baseline: 1.2951x; 1.2951x over previous
import functools

import numpy as np
import jax
import jax.numpy as jnp
from jax.experimental import pallas as pl
from jax.experimental.pallas import tpu as pltpu

_S = 128


def _upsample_matrix(n_in, n_out):
    pos = np.arange(n_out) * (n_in - 1) / (n_out - 1)
    lo = np.clip(np.floor(pos).astype(np.int64), 0, n_in - 2)
    frac = (pos - lo).astype(np.float32)
    m = np.zeros((n_out, n_in), np.float32)
    m[np.arange(n_out), lo] += 1.0 - frac
    m[np.arange(n_out), lo + 1] += frac
    return m


def _block_kernel(x_ref, m_ref, w1_ref, b1_ref, w2_ref, bout_ref, wsc_ref,
                  o_ref, pb1_ref, p1_ref, pb2_ref, p2_ref, *, H, W, C, Ho, Wo, Co):
    x = x_ref[...]
    xcat = jnp.concatenate([jnp.maximum(x, 0.0), x], axis=2)
    m = m_ref[...]
    t = jnp.dot(m, xcat.reshape(H, W * 2 * C),
                preferred_element_type=jnp.float32)
    tt = jnp.transpose(t.reshape(Ho, W, 2 * C), (1, 0, 2))
    u = jnp.dot(m, tt.reshape(W, Ho * 2 * C),
                preferred_element_type=jnp.float32)
    u = u.reshape(Wo, Ho, 2 * C)

    pb1_ref[...] = jnp.zeros_like(pb1_ref)
    pb1_ref[1:1 + Wo, 1:1 + Ho, :] = u[:, :, :C].astype(jnp.bfloat16)
    f1 = pb1_ref[...].reshape((Wo + 3) * _S, C)
    np1 = (Wo + 2) * _S
    for kc in range(3):
        p1_ref[:, kc * C:(kc + 1) * C] = f1[kc:kc + np1, :]
    mo = Wo * _S
    acc = jnp.dot(p1_ref[0:mo], w1_ref[0], preferred_element_type=jnp.float32)
    acc = acc + jnp.dot(p1_ref[_S:_S + mo], w1_ref[1],
                        preferred_element_type=jnp.float32)
    acc = acc + jnp.dot(p1_ref[2 * _S:2 * _S + mo], w1_ref[2],
                        preferred_element_type=jnp.float32)
    h = jnp.maximum(acc + b1_ref[...], 0.0).astype(jnp.bfloat16)

    pb2_ref[...] = jnp.zeros_like(pb2_ref)
    pb2_ref[1:1 + Wo, 1:1 + Ho, :] = h.reshape(Wo, _S, Co)[:, :Ho, :]
    f2 = pb2_ref[...].reshape((Wo + 3) * _S, Co)
    for kc in range(3):
        p2_ref[:, kc * Co:(kc + 1) * Co] = f2[kc:kc + np1, :]
    acc2 = jnp.dot(p2_ref[0:mo], w2_ref[0], preferred_element_type=jnp.float32)
    acc2 = acc2 + jnp.dot(p2_ref[_S:_S + mo], w2_ref[1],
                          preferred_element_type=jnp.float32)
    acc2 = acc2 + jnp.dot(p2_ref[2 * _S:2 * _S + mo], w2_ref[2],
                          preferred_element_type=jnp.float32)

    us = u[:, :, C:].astype(jnp.bfloat16).reshape(Wo * Ho, C)
    sc = jnp.dot(us, wsc_ref[...], preferred_element_type=jnp.float32)
    out = (acc2.reshape(Wo, _S, Co)[:, :Ho, :]
           + sc.reshape(Wo, Ho, Co) + bout_ref[...])
    o_ref[...] = out


def kernel(w1, b1, w2, b2, wsc, bsc, x):
    B, Cin, H, W = x.shape
    Co = w1.shape[-1]
    Ho, Wo = 2 * H, 2 * W
    xh = jnp.transpose(x, (0, 2, 3, 1))
    m = jnp.asarray(_upsample_matrix(H, Ho))
    w1t = jnp.transpose(w1[:, :, :Cin, :], (1, 0, 2, 3)) \
             .reshape(3, 3 * Cin, Co).astype(jnp.bfloat16)
    w2t = jnp.transpose(w2, (1, 0, 2, 3)) \
             .reshape(3, 3 * Co, Co).astype(jnp.bfloat16)
    wsct = wsc[:Cin, :].astype(jnp.bfloat16)
    bout = (b2 + bsc).reshape(1, Co)

    kern = functools.partial(_block_kernel, H=H, W=W, C=Cin, Ho=Ho, Wo=Wo, Co=Co)
    out_t = pl.pallas_call(
        kern,
        out_shape=jax.ShapeDtypeStruct((B, Wo, Ho, Co), jnp.float32),
        grid_spec=pltpu.PrefetchScalarGridSpec(
            num_scalar_prefetch=0,
            grid=(B,),
            in_specs=[
                pl.BlockSpec((None, H, W, Cin), lambda b: (b, 0, 0, 0)),
                pl.BlockSpec((Ho, H), lambda b: (0, 0)),
                pl.BlockSpec((3, 3 * Cin, Co), lambda b: (0, 0, 0)),
                pl.BlockSpec((1, Co), lambda b: (0, 0)),
                pl.BlockSpec((3, 3 * Co, Co), lambda b: (0, 0, 0)),
                pl.BlockSpec((1, Co), lambda b: (0, 0)),
                pl.BlockSpec((Cin, Co), lambda b: (0, 0)),
            ],
            out_specs=pl.BlockSpec((None, Wo, Ho, Co), lambda b: (b, 0, 0, 0)),
            scratch_shapes=[
                pltpu.VMEM(((Wo + 3), _S, Cin), jnp.bfloat16),
                pltpu.VMEM(((Wo + 2) * _S, 3 * Cin), jnp.bfloat16),
                pltpu.VMEM(((Wo + 3), _S, Co), jnp.bfloat16),
                pltpu.VMEM(((Wo + 2) * _S, 3 * Co), jnp.bfloat16),
            ],
        ),
        compiler_params=pltpu.CompilerParams(dimension_semantics=("parallel",)),
    )(xh, m, w1t, b1.reshape(1, Co), w2t, bout, wsct)
    return jnp.transpose(out_t, (0, 3, 2, 1))


# speedup vs baseline: 1.3741x; 1.0610x over previous
import functools

import numpy as np
import jax
import jax.numpy as jnp
from jax.experimental import pallas as pl
from jax.experimental.pallas import tpu as pltpu

_S = 128


def _upsample_matrix(n_in, n_out):
    pos = np.arange(n_out) * (n_in - 1) / (n_out - 1)
    lo = np.clip(np.floor(pos).astype(np.int64), 0, n_in - 2)
    frac = (pos - lo).astype(np.float32)
    m = np.zeros((n_out, n_in), np.float32)
    m[np.arange(n_out), lo] += 1.0 - frac
    m[np.arange(n_out), lo + 1] += frac
    return m


def _block_kernel(x_ref, m_ref, w1_ref, b1_ref, w2_ref, bout_ref, wsc_ref,
                  o_ref, pb1_ref, p1_ref, pb2_ref, p2_ref, *, H, W, C, Ho, Wo, Co):
    x = x_ref[...]
    xcat = jnp.concatenate([jnp.maximum(x, 0.0), x], axis=2)
    m = m_ref[...]
    t = jnp.dot(m, xcat.reshape(H, W * 2 * C),
                preferred_element_type=jnp.float32)
    tt = jnp.transpose(t.reshape(Ho, W, 2 * C), (1, 0, 2))
    u = jnp.dot(m, tt.reshape(W, Ho * 2 * C),
                preferred_element_type=jnp.float32)
    u = u.reshape(Wo, Ho, 2 * C)

    pb1_ref[...] = jnp.zeros_like(pb1_ref)
    pb1_ref[1:1 + Wo, 8:8 + Ho, :] = u[:, :, :C].astype(jnp.bfloat16)
    f1 = pb1_ref[...].reshape((Wo + 3) * _S, C)
    np1 = (Wo + 2) * _S
    for kc in range(3):
        p1_ref[:, kc * C:(kc + 1) * C] = f1[7 + kc:7 + kc + np1, :]
    mo = Wo * _S
    acc = jnp.dot(p1_ref[0:mo], w1_ref[0], preferred_element_type=jnp.float32)
    acc = acc + jnp.dot(p1_ref[_S:_S + mo], w1_ref[1],
                        preferred_element_type=jnp.float32)
    acc = acc + jnp.dot(p1_ref[2 * _S:2 * _S + mo], w1_ref[2],
                        preferred_element_type=jnp.float32)
    h = jnp.maximum(acc + b1_ref[...], 0.0).astype(jnp.bfloat16)

    pb2_ref[...] = jnp.zeros_like(pb2_ref)
    pb2_ref[1:1 + Wo, 8:8 + Ho, :] = h.reshape(Wo, _S, Co)[:, :Ho, :]
    f2 = pb2_ref[...].reshape((Wo + 3) * _S, Co)
    for kc in range(3):
        p2_ref[:, kc * Co:(kc + 1) * Co] = f2[7 + kc:7 + kc + np1, :]
    acc2 = jnp.dot(p2_ref[0:mo], w2_ref[0], preferred_element_type=jnp.float32)
    acc2 = acc2 + jnp.dot(p2_ref[_S:_S + mo], w2_ref[1],
                          preferred_element_type=jnp.float32)
    acc2 = acc2 + jnp.dot(p2_ref[2 * _S:2 * _S + mo], w2_ref[2],
                          preferred_element_type=jnp.float32)

    us = u[:, :, C:].astype(jnp.bfloat16).reshape(Wo * Ho, C)
    sc = jnp.dot(us, wsc_ref[...], preferred_element_type=jnp.float32)
    out = (acc2.reshape(Wo, _S, Co)[:, :Ho, :]
           + sc.reshape(Wo, Ho, Co) + bout_ref[...])
    o_ref[...] = out


def kernel(w1, b1, w2, b2, wsc, bsc, x):
    B, Cin, H, W = x.shape
    Co = w1.shape[-1]
    Ho, Wo = 2 * H, 2 * W
    xh = jnp.transpose(x, (0, 2, 3, 1))
    m = jnp.asarray(_upsample_matrix(H, Ho))
    w1t = jnp.transpose(w1[:, :, :Cin, :], (1, 0, 2, 3)) \
             .reshape(3, 3 * Cin, Co).astype(jnp.bfloat16)
    w2t = jnp.transpose(w2, (1, 0, 2, 3)) \
             .reshape(3, 3 * Co, Co).astype(jnp.bfloat16)
    wsct = wsc[:Cin, :].astype(jnp.bfloat16)
    bout = (b2 + bsc).reshape(1, Co)

    kern = functools.partial(_block_kernel, H=H, W=W, C=Cin, Ho=Ho, Wo=Wo, Co=Co)
    out_t = pl.pallas_call(
        kern,
        out_shape=jax.ShapeDtypeStruct((B, Wo, Ho, Co), jnp.float32),
        grid_spec=pltpu.PrefetchScalarGridSpec(
            num_scalar_prefetch=0,
            grid=(B,),
            in_specs=[
                pl.BlockSpec((None, H, W, Cin), lambda b: (b, 0, 0, 0)),
                pl.BlockSpec((Ho, H), lambda b: (0, 0)),
                pl.BlockSpec((3, 3 * Cin, Co), lambda b: (0, 0, 0)),
                pl.BlockSpec((1, Co), lambda b: (0, 0)),
                pl.BlockSpec((3, 3 * Co, Co), lambda b: (0, 0, 0)),
                pl.BlockSpec((1, Co), lambda b: (0, 0)),
                pl.BlockSpec((Cin, Co), lambda b: (0, 0)),
            ],
            out_specs=pl.BlockSpec((None, Wo, Ho, Co), lambda b: (b, 0, 0, 0)),
            scratch_shapes=[
                pltpu.VMEM(((Wo + 3), _S, Cin), jnp.bfloat16),
                pltpu.VMEM(((Wo + 2) * _S, 3 * Cin), jnp.bfloat16),
                pltpu.VMEM(((Wo + 3), _S, Co), jnp.bfloat16),
                pltpu.VMEM(((Wo + 2) * _S, 3 * Co), jnp.bfloat16),
            ],
        ),
        compiler_params=pltpu.CompilerParams(dimension_semantics=("parallel",)),
    )(xh, m, w1t, b1.reshape(1, Co), w2t, bout, wsct)
    return jnp.transpose(out_t, (0, 3, 2, 1))
